# baseline (speedup 1.0000x reference)
import functools

import jax
import jax.numpy as jnp
from jax import lax
from jax.experimental import pallas as pl
from jax.experimental.pallas import tpu as pltpu
from jax.experimental.pallas import tpu_sc as plsc

BATCH = 4096
SEQ = 200
HIDDEN = 64
N = BATCH * SEQ

_info = plsc.get_sparse_core_info()
NC, NS = _info.num_cores, _info.num_subcores
NW = NC * NS
PER_W = N // NW
CHUNK = 128
STEPS = PER_W // CHUNK


@functools.partial(
    pl.kernel,
    out_type=jax.ShapeDtypeStruct((N, HIDDEN), jnp.float32),
    mesh=plsc.VectorSubcoreMesh(core_axis_name="c", subcore_axis_name="s"),
    scratch_types=[
        pltpu.VMEM((STEPS, CHUNK), jnp.int32),
        pltpu.VMEM((CHUNK, HIDDEN), jnp.float32),
        pltpu.SemaphoreType.DMA,
    ],
    compiler_params=pltpu.CompilerParams(use_tc_tiling_on_sc=False),
)
def _gather_kernel(table_hbm, idx_hbm, out_hbm, idx_v, rows_v, sem):
    wid = lax.axis_index("s") * NC + lax.axis_index("c")
    pltpu.sync_copy(idx_hbm.at[wid], idx_v)
    base = wid * PER_W

    def step(i, carry):
        pltpu.async_copy(table_hbm.at[idx_v.at[i]], rows_v, sem).wait()
        pltpu.sync_copy(rows_v, out_hbm.at[pl.ds(base + i * CHUNK, CHUNK)])
        return carry

    lax.fori_loop(0, STEPS, step, 0)


def kernel(inp_seq, inp_seq_len, embedding_table):
    del inp_seq_len
    idx = inp_seq.reshape(NW, STEPS, CHUNK).astype(jnp.int32)
    out = _gather_kernel(embedding_table, idx)
    return out.reshape(BATCH, SEQ, HIDDEN)


# speedup vs baseline: 1.1124x; 1.1124x over previous
import functools

import jax
import jax.numpy as jnp
from jax import lax
from jax.experimental import pallas as pl
from jax.experimental.pallas import tpu as pltpu
from jax.experimental.pallas import tpu_sc as plsc

BATCH = 4096
SEQ = 200
HIDDEN = 64
N = BATCH * SEQ

_info = plsc.get_sparse_core_info()
NC, NS = _info.num_cores, _info.num_subcores
NW = NC * NS
PER_W = N // NW
CHUNK = 128
STEPS = PER_W // CHUNK
NBUF = 4
GROUPS = STEPS // NBUF


@functools.partial(
    pl.kernel,
    out_type=jax.ShapeDtypeStruct((N, HIDDEN), jnp.float32),
    mesh=plsc.VectorSubcoreMesh(core_axis_name="c", subcore_axis_name="s"),
    scratch_types=[
        pltpu.VMEM((STEPS, CHUNK), jnp.int32),
        pltpu.VMEM((NBUF, CHUNK, HIDDEN), jnp.float32),
        pltpu.SemaphoreType.DMA((NBUF,)),
        pltpu.SemaphoreType.DMA((NBUF,)),
    ],
    compiler_params=pltpu.CompilerParams(use_tc_tiling_on_sc=False),
)
def _gather_kernel(table_hbm, idx_hbm, out_hbm, idx_v, rows_v, gsem, wsem):
    wid = lax.axis_index("s") * NC + lax.axis_index("c")
    pltpu.sync_copy(idx_hbm.at[wid], idx_v)
    base = wid * PER_W

    def gather(i, b):
        return pltpu.make_async_copy(
            table_hbm.at[idx_v.at[i]], rows_v.at[b], gsem.at[b])

    def writeback(i, b):
        return pltpu.make_async_copy(
            rows_v.at[b], out_hbm.at[pl.ds(base + i * CHUNK, CHUNK)], wsem.at[b])

    for b in range(NBUF):
        gather(b, b).start()

    def group(g, carry):
        i0 = g * NBUF
        for b in range(NBUF):
            gather(i0 + b, b).wait()
            writeback(i0 + b, b).start()
        for b in range(NBUF):
            writeback(i0 + b, b).wait()

            @pl.when(g + 1 < GROUPS)
            def _():
                gather(i0 + NBUF + b, b).start()

        return carry

    lax.fori_loop(0, GROUPS, group, 0)


def kernel(inp_seq, inp_seq_len, embedding_table):
    del inp_seq_len
    idx = inp_seq.reshape(NW, STEPS, CHUNK).astype(jnp.int32)
    out = _gather_kernel(embedding_table, idx)
    return out.reshape(BATCH, SEQ, HIDDEN)


# speedup vs baseline: 1.1144x; 1.0018x over previous
import functools

import jax
import jax.numpy as jnp
from jax import lax
from jax.experimental import pallas as pl
from jax.experimental.pallas import tpu as pltpu
from jax.experimental.pallas import tpu_sc as plsc

BATCH = 4096
SEQ = 200
HIDDEN = 64

_info = plsc.get_sparse_core_info()
NC, NS = _info.num_cores, _info.num_subcores
NW = NC * NS
B_PER_W = BATCH // NW
CHUNK = 128
REM = SEQ - CHUNK
NBUF = 4
GROUPS = B_PER_W // NBUF


@functools.partial(
    pl.kernel,
    out_type=jax.ShapeDtypeStruct((BATCH, SEQ, HIDDEN), jnp.float32),
    mesh=plsc.VectorSubcoreMesh(core_axis_name="c", subcore_axis_name="s"),
    scratch_types=[
        pltpu.VMEM((B_PER_W, SEQ), jnp.int32),
        pltpu.VMEM((NBUF, SEQ, HIDDEN), jnp.float32),
        pltpu.SemaphoreType.DMA((NBUF,)),
        pltpu.SemaphoreType.DMA((NBUF,)),
    ],
    compiler_params=pltpu.CompilerParams(use_tc_tiling_on_sc=False),
)
def _gather_kernel(table_hbm, idx_hbm, out_hbm, idx_v, rows_v, gsem, wsem):
    wid = lax.axis_index("s") * NC + lax.axis_index("c")
    base = wid * B_PER_W
    pltpu.sync_copy(idx_hbm.at[pl.ds(base, B_PER_W)], idx_v)

    def gathers(i, b):
        return (
            pltpu.make_async_copy(
                table_hbm.at[idx_v.at[i, pl.ds(0, CHUNK)]],
                rows_v.at[b, pl.ds(0, CHUNK)], gsem.at[b]),
            pltpu.make_async_copy(
                table_hbm.at[idx_v.at[i, pl.ds(CHUNK, REM)]],
                rows_v.at[b, pl.ds(CHUNK, REM)], gsem.at[b]),
        )

    def writeback(i, b):
        return pltpu.make_async_copy(rows_v.at[b], out_hbm.at[base + i], wsem.at[b])

    for b in range(NBUF):
        for c in gathers(b, b):
            c.start()

    def group(g, carry):
        i0 = g * NBUF
        for b in range(NBUF):
            for c in gathers(i0 + b, b):
                c.wait()
            writeback(i0 + b, b).start()
        for b in range(NBUF):
            writeback(i0 + b, b).wait()

            @pl.when(g + 1 < GROUPS)
            def _():
                for c in gathers(i0 + NBUF + b, b):
                    c.start()

        return carry

    lax.fori_loop(0, GROUPS, group, 0)


def kernel(inp_seq, inp_seq_len, embedding_table):
    del inp_seq_len
    return _gather_kernel(embedding_table, inp_seq.astype(jnp.int32))
